# baseline (speedup 1.0000x reference)
import functools

import jax
import jax.numpy as jnp
from jax import lax
from jax.experimental import pallas as pl
from jax.experimental.pallas import tpu as pltpu
from jax.experimental.pallas import tpu_sc as plsc

N = 10000
H = 128
G = 64
L = 3
OUT = 10

TILES = 32
NB = 32
BSZ = 320
NPAD = TILES * BSZ
EPG = 128
GPT = 80
CHUNK = GPT * EPG
EPAD = TILES * CHUNK
E_BIN = 331776
SCG = 32
NBUF = 4
E_BIN2 = E_BIN + SCG * EPG

ROW_BLK = 1000
POOL_BLK = 400
PRECISION = lax.Precision.HIGHEST

_mesh = plsc.VectorSubcoreMesh(core_axis_name="c", subcore_axis_name="s")
_SC_PARAMS = pltpu.CompilerParams(needs_layout_passes=False)


def _tile_id():
    return lax.axis_index("c") * 16 + lax.axis_index("s")


def _lanes():
    return lax.iota(jnp.int32, 16)


def _div_bsz(d):
    return lax.shift_right_logical(d * 3277, 20)


def _hist_body(dstg_hbm, out_hbm, dst_v, cnt2):
    t = _tile_id()
    pltpu.sync_copy(dstg_hbm.at[t], dst_v)
    for k in range(NB):
        cnt2[pl.ds(16 * k, 16)] = jnp.zeros((16,), jnp.int32)
    lanes = _lanes()
    ones = jnp.ones((16,), jnp.int32)

    def body(g, carry):
        for j in range(8):
            d = dst_v[g, pl.ds(16 * j, 16)]
            b = _div_bsz(d)
            plsc.addupdate_scatter(cnt2, [lanes * NB + b], ones)
        return carry

    lax.fori_loop(0, GPT, body, 0)
    pltpu.sync_copy(cnt2, out_hbm.at[t])


_hist = pl.kernel(
    _hist_body,
    out_type=jax.ShapeDtypeStruct((TILES, 16 * NB), jnp.int32),
    mesh=_mesh,
    compiler_params=_SC_PARAMS,
    scratch_types=[
        pltpu.VMEM((GPT, EPG), jnp.int32),
        pltpu.VMEM((16 * NB,), jnp.int32),
    ],
)


def _binscatter_body(srcg_hbm, dstg_hbm, off3_hbm, padinfo_hbm,
                     bsrc_hbm, bdstl_hbm,
                     src_v, dst_v, cursor, slot_st, dstl_st, zsrc,
                     sem_a, sem_b):
    t = _tile_id()
    pltpu.sync_copy(srcg_hbm.at[t], src_v)
    pltpu.sync_copy(dstg_hbm.at[t], dst_v)
    pltpu.sync_copy(off3_hbm.at[t], cursor)
    lanes = _lanes()

    def body(g, carry):
        for j in range(8):
            sl = pl.ds(16 * j, 16)
            d = dst_v[g, sl]
            b = _div_bsz(d)
            cidx = lanes * NB + b
            cur = plsc.load_gather(cursor, [cidx])
            plsc.store_scatter(cursor, [cidx], cur + 1)
            slot_st[g, sl] = cur
            dstl_st[g, sl] = d - b * BSZ
        return carry

    lax.fori_loop(0, GPT, body, 0)

    pltpu.sync_copy(padinfo_hbm.at[t], zsrc)
    pv = zsrc[pl.ds(0, 16)]
    ps = pv[0]
    pc = pv[1]
    for k in range(8):
        sl = pl.ds(16 * k, 16)
        pos = k * 16 + lanes
        slot_st[GPT, sl] = jnp.where(pos < pc, ps + pos, E_BIN + pos)
        dstl_st[GPT, sl] = jnp.full((16,), BSZ, jnp.int32)
    for k in range(8):
        zsrc[pl.ds(16 * k, 16)] = jnp.zeros((16,), jnp.int32)

    for j0 in range(0, GPT + 1, 16):
        descs = []
        for j in range(j0, min(j0 + 16, GPT + 1)):
            sref = zsrc if j == GPT else src_v.at[j]
            descs.append(pltpu.async_copy(
                sref, bsrc_hbm.at[slot_st.at[j]], sem_a))
            descs.append(pltpu.async_copy(
                dstl_st.at[j], bdstl_hbm.at[slot_st.at[j]], sem_b))
        for d in descs:
            d.wait()


_binscatter = pl.kernel(
    _binscatter_body,
    out_type=(jax.ShapeDtypeStruct((E_BIN2,), jnp.int32),
              jax.ShapeDtypeStruct((E_BIN2,), jnp.int32)),
    mesh=_mesh,
    compiler_params=_SC_PARAMS,
    scratch_types=[
        pltpu.VMEM((GPT, EPG), jnp.int32),
        pltpu.VMEM((GPT, EPG), jnp.int32),
        pltpu.VMEM((16 * NB,), jnp.int32),
        pltpu.VMEM((GPT + 1, EPG), jnp.int32),
        pltpu.VMEM((GPT + 1, EPG), jnp.int32),
        pltpu.VMEM((EPG,), jnp.int32),
        pltpu.SemaphoreType.DMA,
        pltpu.SemaphoreType.DMA,
    ],
)


def _agg_body(h_hbm, bsrc_hbm, bdstl_hbm, meta_hbm, part_hbm,
              src8, dstl8, rows0, rows1, rows2, rows3, acc, meta_v,
              sem0, sem1, sem2, sem3):
    t = _tile_id()
    pltpu.sync_copy(meta_hbm.at[t], meta_v)
    z = jnp.zeros((16,), jnp.float32)
    lanes = _lanes()

    def zbody(i, carry):
        for u in range(8):
            acc[pl.ds(i * 128 + u * 16, 16)] = z
        return carry

    lax.fori_loop(0, (BSZ + 1) * H // 128, zbody, 0)
    mv = meta_v[pl.ds(0, 16)]
    g0 = mv[0]
    ng = mv[1]
    rows = (rows0, rows1, rows2, rows3)
    sems = (sem0, sem1, sem2, sem3)

    def start_gather(k, b):
        return pltpu.async_copy(
            h_hbm.at[src8.at[pl.ds(k * EPG, EPG)]], rows[b], sems[b])

    def accumulate(rbuf, k):
        def qbody(q, cc):
            dv = dstl8[pl.ds(k * EPG + q * 16, 16)]
            for r in range(16):
                e = q * 16 + r
                idx = dv[r] * H + lanes
                for kk in range(8):
                    vals = rbuf[e, pl.ds(16 * kk, 16)]
                    plsc.addupdate_scatter(acc, [idx], vals)
                    if kk < 7:
                        idx = idx + 16
            return cc

        lax.fori_loop(0, EPG // 16, qbody, 0)

    nsc = (ng + SCG - 1) // SCG

    def outer(i, carry):
        jb = i * SCG
        base = (g0 + jb) * EPG
        pltpu.sync_copy(bsrc_hbm.at[pl.ds(base, SCG * EPG)], src8)
        pltpu.sync_copy(bdstl_hbm.at[pl.ds(base, SCG * EPG)], dstl8)
        for b in range(NBUF):
            @pl.when(jb + b < ng)
            def _(b=b):
                start_gather(b, b)

        def middle(m, cc):
            for b in range(NBUF):
                k = m * NBUF + b
                j = jb + k

                @pl.when(j < ng)
                def _(k=k, b=b, j=j):
                    pltpu.make_async_copy(
                        h_hbm.at[src8.at[pl.ds(k * EPG, EPG)]],
                        rows[b], sems[b]).wait()
                    accumulate(rows[b], k)

                    @pl.when(jnp.logical_and(m < SCG // NBUF - 1,
                                             j + NBUF < ng))
                    def _():
                        start_gather(k + NBUF, b)
            return cc

        lax.fori_loop(0, SCG // NBUF, middle, 0)
        return carry

    lax.fori_loop(0, nsc, outer, 0)
    pltpu.sync_copy(acc.at[pl.ds(0, BSZ * H)],
                    part_hbm.at[pl.ds(t * BSZ * H, BSZ * H)])


_aggregate = pl.kernel(
    _agg_body,
    out_type=jax.ShapeDtypeStruct((NPAD * H,), jnp.float32),
    mesh=_mesh,
    compiler_params=_SC_PARAMS,
    scratch_types=[
        pltpu.VMEM((SCG * EPG,), jnp.int32),
        pltpu.VMEM((SCG * EPG,), jnp.int32),
        pltpu.VMEM((EPG, H), jnp.float32),
        pltpu.VMEM((EPG, H), jnp.float32),
        pltpu.VMEM((EPG, H), jnp.float32),
        pltpu.VMEM((EPG, H), jnp.float32),
        pltpu.VMEM(((BSZ + 1) * H,), jnp.float32),
        pltpu.VMEM((16,), jnp.int32),
        pltpu.SemaphoreType.DMA,
        pltpu.SemaphoreType.DMA,
        pltpu.SemaphoreType.DMA,
        pltpu.SemaphoreType.DMA,
    ],
)


def _embed_body(x_ref, p_ref, wx_ref, wp_ref, b_ref, o_ref):
    acc = jnp.dot(x_ref[...], wx_ref[...], precision=PRECISION)
    acc += jnp.dot(p_ref[...], wp_ref[...], precision=PRECISION)
    o_ref[...] = acc + b_ref[...]


def _embed(x, pos_p, wxt, wpt, brow):
    grid = N // ROW_BLK
    return pl.pallas_call(
        _embed_body,
        grid=(grid,),
        in_specs=[
            pl.BlockSpec((ROW_BLK, H), lambda i: (i, 0)),
            pl.BlockSpec((ROW_BLK, H), lambda i: (i, 0)),
            pl.BlockSpec((H, H), lambda i: (0, 0)),
            pl.BlockSpec((H, H), lambda i: (0, 0)),
            pl.BlockSpec((1, H), lambda i: (0, 0)),
        ],
        out_specs=pl.BlockSpec((ROW_BLK, H), lambda i: (i, 0)),
        out_shape=jax.ShapeDtypeStruct((N, H), jnp.float32),
    )(x, pos_p, wxt, wpt, brow)


def _mlp_body(h_ref, part_ref, sc_ref, w1_ref, b1_ref, w2_ref, b2_ref, o_ref):
    hin = h_ref[...] * sc_ref[...] + part_ref[...]
    y = jnp.dot(hin, w1_ref[...], precision=PRECISION) + b1_ref[...]
    y = jnp.maximum(y, 0.0)
    z = jnp.dot(y, w2_ref[...], precision=PRECISION) + b2_ref[...]
    o_ref[...] = jnp.maximum(z, 0.0)


def _mlp(h, part, scale_row, w1t, b1row, w2t, b2row):
    grid = N // ROW_BLK
    return pl.pallas_call(
        _mlp_body,
        grid=(grid,),
        in_specs=[
            pl.BlockSpec((ROW_BLK, H), lambda i: (i, 0)),
            pl.BlockSpec((ROW_BLK, H), lambda i: (i, 0)),
            pl.BlockSpec((1, H), lambda i: (0, 0)),
            pl.BlockSpec((H, H), lambda i: (0, 0)),
            pl.BlockSpec((1, H), lambda i: (0, 0)),
            pl.BlockSpec((H, H), lambda i: (0, 0)),
            pl.BlockSpec((1, H), lambda i: (0, 0)),
        ],
        out_specs=pl.BlockSpec((ROW_BLK, H), lambda i: (i, 0)),
        out_shape=jax.ShapeDtypeStruct((N, H), jnp.float32),
    )(h, part, scale_row, w1t, b1row, w2t, b2row)


def _pool_body(h_ref, b3_ref, w1_ref, b1_ref, w2_ref, b2_ref, o_ref, acc_ref):
    step = pl.program_id(0)

    @pl.when(step == 0)
    def _():
        acc_ref[...] = jnp.full((G, H), -jnp.inf, jnp.float32)

    bb = b3_ref[0]
    hb = h_ref[...]
    for g in range(G):
        m = jnp.max(jnp.where(bb == g, hb, -jnp.inf), axis=0)
        acc_ref[g, :] = jnp.maximum(acc_ref[g, :], m)

    @pl.when(step == pl.num_programs(0) - 1)
    def _():
        pooled = acc_ref[...]
        y = jnp.dot(pooled, w1_ref[...], precision=PRECISION) + b1_ref[...]
        y = jnp.maximum(y, 0.0)
        o_ref[...] = jnp.dot(y, w2_ref[...], precision=PRECISION) + b2_ref[...]


def _pool_head(h, batch3, w1t, b1row, w2tp, b2row):
    grid = N // POOL_BLK
    return pl.pallas_call(
        _pool_body,
        grid=(grid,),
        in_specs=[
            pl.BlockSpec((POOL_BLK, H), lambda i: (i, 0)),
            pl.BlockSpec((1, POOL_BLK, 1), lambda i: (i, 0, 0)),
            pl.BlockSpec((H, H), lambda i: (0, 0)),
            pl.BlockSpec((1, H), lambda i: (0, 0)),
            pl.BlockSpec((H, H), lambda i: (0, 0)),
            pl.BlockSpec((1, H), lambda i: (0, 0)),
        ],
        out_specs=pl.BlockSpec((G, H), lambda i: (0, 0)),
        out_shape=jax.ShapeDtypeStruct((G, H), jnp.float32),
        scratch_shapes=[pltpu.VMEM((G, H), jnp.float32)],
    )(h, batch3, w1t, b1row, w2tp, b2row)


def kernel(x, pos, edge_index, batch, W_emb, b_emb, eps, Wc1, bc1, g1, be1,
           m1, v1, Wc2, bc2, g2, be2, m2, v2, W1, b1, W2, b2):
    E = edge_index.shape[1]
    src = edge_index[0]
    dst = edge_index[1]
    srcg = jnp.concatenate(
        [src, jnp.zeros((EPAD - E,), jnp.int32)]).reshape(TILES, GPT, EPG)
    dstg = jnp.concatenate(
        [dst, jnp.full((EPAD - E,), N, jnp.int32)]).reshape(TILES, GPT, EPG)

    cnt = _hist(dstg)
    cnt3 = cnt.reshape(TILES, 16, NB)
    tot_b = jnp.sum(cnt3, axis=(0, 1))
    totp_b = ((tot_b + (EPG - 1)) // EPG) * EPG
    csum = jnp.cumsum(totp_b)
    bstart = jnp.concatenate([jnp.zeros((1,), jnp.int32), csum[:-1]])
    c_btl = cnt3.transpose(2, 0, 1).reshape(NB, TILES * 16)
    excl = jnp.cumsum(c_btl, axis=1) - c_btl
    off3 = (bstart[:, None] + excl).reshape(NB, TILES, 16).transpose(1, 2, 0)
    off3 = (off3.reshape(TILES, 16 * NB)).astype(jnp.int32)
    padstart = (bstart + tot_b).astype(jnp.int32)
    padcnt = (totp_b - tot_b).astype(jnp.int32)
    padinfo = jnp.zeros((TILES, EPG), jnp.int32)
    padinfo = padinfo.at[:, 0].set(padstart).at[:, 1].set(padcnt)
    bsrc, bdstl = _binscatter(srcg, dstg, off3, padinfo)
    meta = jnp.zeros((TILES, 16), jnp.int32)
    meta = meta.at[:, 0].set((bstart // EPG).astype(jnp.int32))
    meta = meta.at[:, 1].set((totp_b // EPG).astype(jnp.int32))

    pos_p = jnp.pad(pos, ((0, 0), (0, H - pos.shape[1])))
    wxt = W_emb[:, :H].T
    wpt = jnp.pad(W_emb[:, H:].T, ((0, H - (W_emb.shape[1] - H)), (0, 0)))
    h = _embed(x, pos_p, wxt, wpt, b_emb.reshape(1, H))

    s1 = g1 / jnp.sqrt(v1 + 1e-5)
    s2 = g2 / jnp.sqrt(v2 + 1e-5)
    w1f = Wc1 * s1[:, :, None]
    b1f = (bc1 - m1) * s1 + be1
    w2f = Wc2 * s2[:, :, None]
    b2f = (bc2 - m2) * s2 + be2

    for i in range(L):
        part = _aggregate(h, bsrc, bdstl, meta).reshape(NPAD, H)
        h = _mlp(h, part,
                 jnp.full((1, H), 1.0 + eps[i], jnp.float32),
                 w1f[i].T, b1f[i].reshape(1, H),
                 w2f[i].T, b2f[i].reshape(1, H))

    batch3 = batch.reshape(N // POOL_BLK, POOL_BLK, 1)
    w2tp = jnp.pad(W2.T, ((0, 0), (0, H - OUT)))
    b2p = jnp.pad(b2, (0, H - OUT)).reshape(1, H)
    out = _pool_head(h, batch3, W1.T, b1.reshape(1, H), w2tp, b2p)
    return out[:, :OUT]
